# baseline (speedup 1.0000x reference)
import jax
import jax.numpy as jnp
from jax.experimental import pallas as pl


def _gin_block(x, src, dst, W, b, g, be):
    agg = jnp.zeros(x.shape, x.dtype).at[dst].add(x[src])
    h = x + agg
    h = jax.nn.relu(h @ W + b)
    mu = jnp.mean(h, axis=0)
    var = jnp.var(h, axis=0)
    h = (h - mu) / jnp.sqrt(var + 1e-5) * g + be
    return h


def _tail_kernel(pooled_ref, wf_ref, bf_ref, out_ref):
    out_ref[...] = jax.nn.relu(
        jnp.dot(pooled_ref[...], wf_ref[...], preferred_element_type=jnp.float32)
        + bf_ref[...]
    )


def kernel(x, edge_index, batch, W1, b1, g1, be1, W2, b2, g2, be2, W3, b3, g3, be3, W4, b4, g4, be4, W5, b5, g5, be5, Wf, bf):
    src = edge_index[0]
    dst = edge_index[1]
    h = _gin_block(x, src, dst, W1, b1, g1, be1)
    h = _gin_block(h, src, dst, W2, b2, g2, be2)
    h = _gin_block(h, src, dst, W3, b3, g3, be3)
    h = _gin_block(h, src, dst, W4, b4, g4, be4)
    h = _gin_block(h, src, dst, W5, b5, g5, be5)
    pooled = jax.ops.segment_sum(h, batch, num_segments=128)
    out = pl.pallas_call(
        _tail_kernel,
        out_shape=jax.ShapeDtypeStruct((128, Wf.shape[1]), jnp.float32),
    )(pooled, Wf, bf.reshape(1, -1))
    return out


# speedup vs baseline: 2.0985x; 2.0985x over previous
import functools

import jax
import jax.numpy as jnp
from jax import lax
from jax.experimental import pallas as pl
from jax.experimental.pallas import tpu as pltpu
from jax.experimental.pallas import tpu_sc as plsc

NN = 10000
EE = 160000
DIN = 256
HH = 1024
GG = 128
LC = 128
E_PAD = 163840
EPT = E_PAD // 16
EB = 128
NPAIR = EPT // (2 * EB)
N_ACC = 10240
RPT = N_ACC // 16
ROW_TILE = 1000
NT = NN // ROW_TILE


@functools.lru_cache(maxsize=None)
def _make_sc_agg(C):
    assert C % 2 == 0
    mesh = plsc.VectorSubcoreMesh(core_axis_name="c", subcore_axis_name="s")

    @functools.partial(
        pl.kernel,
        out_type=jax.ShapeDtypeStruct((C * NN, LC), jnp.float32),
        mesh=mesh,
        scratch_types=[
            pltpu.VMEM_SHARED((N_ACC, LC), jnp.float32),
            pltpu.VMEM((EB,), jnp.int32),
            pltpu.VMEM((EB,), jnp.int32),
            pltpu.VMEM((EB,), jnp.int32),
            pltpu.VMEM((EB,), jnp.int32),
            pltpu.VMEM((EB, LC), jnp.float32),
            pltpu.VMEM((EB, LC), jnp.float32),
            pltpu.SemaphoreType.DMA,
            pltpu.SemaphoreType.DMA,
        ],
    )
    def sc_agg(z_hbm, srcc_hbm, dst_hbm, zero_hbm, out_hbm,
               acc, sidx0, sidx1, didx0, didx1, rows0, rows1, gsem0, gsem1):
        core = lax.axis_index("c")
        tid = lax.axis_index("s")
        ebase = tid * EPT

        for kk in range(C // 2):
            j = kk * 2 + core
            soff = j * E_PAD

            pltpu.sync_copy(zero_hbm, acc.at[pl.ds(tid * RPT, RPT)])
            plsc.subcore_barrier()

            pltpu.sync_copy(srcc_hbm.at[pl.ds(soff + ebase, EB)], sidx0)
            pltpu.sync_copy(dst_hbm.at[pl.ds(ebase, EB)], didx0)
            pltpu.async_copy(z_hbm.at[sidx0], rows0, gsem0)

            def body(pair, carry):
                i0 = ebase + 2 * pair * EB
                pltpu.sync_copy(srcc_hbm.at[pl.ds(soff + i0 + EB, EB)], sidx1)
                pltpu.sync_copy(dst_hbm.at[pl.ds(i0 + EB, EB)], didx1)
                pltpu.async_copy(z_hbm.at[sidx1], rows1, gsem1)
                pltpu.make_async_copy(z_hbm.at[sidx0], rows0, gsem0).wait()
                pltpu.sync_copy(rows0, acc.at[didx0], add=True)

                @pl.when(pair + 1 < NPAIR)
                def _():
                    pltpu.sync_copy(
                        srcc_hbm.at[pl.ds(soff + i0 + 2 * EB, EB)], sidx0)
                    pltpu.sync_copy(dst_hbm.at[pl.ds(i0 + 2 * EB, EB)], didx0)
                    pltpu.async_copy(z_hbm.at[sidx0], rows0, gsem0)

                pltpu.make_async_copy(z_hbm.at[sidx1], rows1, gsem1).wait()
                pltpu.sync_copy(rows1, acc.at[didx1], add=True)
                return carry

            lax.fori_loop(0, NPAIR, body, 0)
            plsc.subcore_barrier()

            obase = j * NN + tid * RPT

            @pl.when(tid < 15)
            def _():
                pltpu.sync_copy(acc.at[pl.ds(tid * RPT, RPT)],
                                out_hbm.at[pl.ds(obase, RPT)])

            @pl.when(tid == 15)
            def _():
                pltpu.sync_copy(acc.at[pl.ds(15 * RPT, NN - 15 * RPT)],
                                out_hbm.at[pl.ds(obase, NN - 15 * RPT)])

    return sc_agg


def _sc_agg_call(zflat, src_c, dst_p, zero_slab, C):
    return _make_sc_agg(C)(zflat, src_c, dst_p, zero_slab)


def _m1_body(x_ref, ax_ref, w_ref, out_ref):
    x = x_ref[...]
    acc = jnp.zeros((ROW_TILE, HH), jnp.float32)
    for j in range(DIN // LC):
        hj = x[:, j * LC:(j + 1) * LC] + ax_ref[j]
        acc = acc + jnp.dot(hj.astype(jnp.bfloat16),
                            w_ref[j * LC:(j + 1) * LC, :].astype(jnp.bfloat16),
                            preferred_element_type=jnp.float32)
    for j in range(HH // LC):
        out_ref[j] = acc[:, j * LC:(j + 1) * LC]


def _mm_body(v_ref, s_ref, t_ref, w_ref, out_ref):
    hh = v_ref[...] * s_ref[...] + t_ref[...]
    z = jnp.dot(hh.astype(jnp.bfloat16), w_ref[...].astype(jnp.bfloat16),
                preferred_element_type=jnp.float32)
    for j in range(HH // LC):
        out_ref[j] = z[:, j * LC:(j + 1) * LC]


def _make_ep_body(use_agg):
    def body(*refs):
        if use_agg:
            (z_ref, a_ref, b_ref, g_ref, be_ref,
             v_ref, s_ref, t_ref, ssum, ssq) = refs
        else:
            (z_ref, b_ref, g_ref, be_ref,
             v_ref, s_ref, t_ref, ssum, ssq) = refs
        ti = pl.program_id(0)

        @pl.when(ti == 0)
        def _():
            ssum[...] = jnp.zeros_like(ssum)
            ssq[...] = jnp.zeros_like(ssq)

        for j in range(HH // LC):
            u = z_ref[j] + b_ref[j][None, :]
            if use_agg:
                u = u + a_ref[j]
            vj = jnp.maximum(u, 0.0)
            v_ref[:, j * LC:(j + 1) * LC] = vj
            ssum[j:j + 1, :] = ssum[j:j + 1, :] + jnp.sum(vj, axis=0,
                                                          keepdims=True)
            ssq[j:j + 1, :] = ssq[j:j + 1, :] + jnp.sum(vj * vj, axis=0,
                                                        keepdims=True)

        @pl.when(ti == NT - 1)
        def _():
            mu = ssum[...] * (1.0 / NN)
            var = ssq[...] * (1.0 / NN) - mu * mu
            sc = g_ref[...] * lax.rsqrt(var + 1e-5)
            s_ref[...] = sc
            t_ref[...] = be_ref[...] - mu * sc

    return body


def _pool_body(v_ref, s_ref, t_ref, bt_ref, wf_ref, bf_ref, out_ref, acc):
    ti = pl.program_id(0)

    @pl.when(ti == 0)
    def _():
        acc[...] = jnp.zeros_like(acc)

    hh = v_ref[...] * s_ref[...] + t_ref[...]
    bt = bt_ref[0, 0, :]
    oneh = (bt[None, :] == lax.broadcasted_iota(
        jnp.int32, (GG, ROW_TILE), 0)).astype(jnp.float32)
    acc[...] = acc[...] + jnp.dot(oneh, hh, preferred_element_type=jnp.float32, precision=lax.Precision.HIGHEST)

    @pl.when(ti == NT - 1)
    def _():
        out_ref[...] = jnp.maximum(
            jnp.dot(acc[...].astype(jnp.bfloat16),
                    wf_ref[...].astype(jnp.bfloat16),
                    preferred_element_type=jnp.float32)
            + bf_ref[...], 0.0)


_ARB = pltpu.CompilerParams(dimension_semantics=("arbitrary",))


def _call_m1(x, aggx3, W1):
    return pl.pallas_call(
        _m1_body,
        grid=(NT,),
        in_specs=[
            pl.BlockSpec((ROW_TILE, DIN), lambda t: (t, 0)),
            pl.BlockSpec((DIN // LC, ROW_TILE, LC), lambda t: (0, t, 0)),
            pl.BlockSpec((DIN, HH), lambda t: (0, 0)),
        ],
        out_specs=pl.BlockSpec((HH // LC, ROW_TILE, LC), lambda t: (0, t, 0)),
        out_shape=jax.ShapeDtypeStruct((HH // LC, NN, LC), jnp.float32),
        compiler_params=_ARB,
    )(x, aggx3, W1)


def _call_mm(v, s, t, W):
    return pl.pallas_call(
        _mm_body,
        grid=(NT,),
        in_specs=[
            pl.BlockSpec((ROW_TILE, HH), lambda i: (i, 0)),
            pl.BlockSpec((1, HH), lambda i: (0, 0)),
            pl.BlockSpec((1, HH), lambda i: (0, 0)),
            pl.BlockSpec((HH, HH), lambda i: (0, 0)),
        ],
        out_specs=pl.BlockSpec((HH // LC, ROW_TILE, LC), lambda i: (0, i, 0)),
        out_shape=jax.ShapeDtypeStruct((HH // LC, NN, LC), jnp.float32),
        compiler_params=_ARB,
    )(v, s, t, W)


def _call_ep(z3, a3, b, g, be):
    use_agg = a3 is not None
    chunk_spec = pl.BlockSpec((HH // LC, ROW_TILE, LC), lambda i: (0, i, 0))
    vec_spec = pl.BlockSpec((HH // LC, LC), lambda i: (0, 0))
    in_specs = [chunk_spec] + ([chunk_spec] if use_agg else []) + [vec_spec] * 3
    args = (z3,) + ((a3,) if use_agg else ()) + (b, g, be)
    return pl.pallas_call(
        _make_ep_body(use_agg),
        grid=(NT,),
        in_specs=in_specs,
        out_specs=[
            pl.BlockSpec((ROW_TILE, HH), lambda i: (i, 0)),
            pl.BlockSpec((HH // LC, LC), lambda i: (0, 0)),
            pl.BlockSpec((HH // LC, LC), lambda i: (0, 0)),
        ],
        out_shape=[
            jax.ShapeDtypeStruct((NN, HH), jnp.float32),
            jax.ShapeDtypeStruct((HH // LC, LC), jnp.float32),
            jax.ShapeDtypeStruct((HH // LC, LC), jnp.float32),
        ],
        scratch_shapes=[
            pltpu.VMEM((HH // LC, LC), jnp.float32),
            pltpu.VMEM((HH // LC, LC), jnp.float32),
        ],
        compiler_params=_ARB,
    )(*args)


def _call_pool(v, s, t, batch3, Wf, bf):
    return pl.pallas_call(
        _pool_body,
        grid=(NT,),
        in_specs=[
            pl.BlockSpec((ROW_TILE, HH), lambda i: (i, 0)),
            pl.BlockSpec((1, HH), lambda i: (0, 0)),
            pl.BlockSpec((1, HH), lambda i: (0, 0)),
            pl.BlockSpec((1, 1, ROW_TILE), lambda i: (i, 0, 0)),
            pl.BlockSpec((HH, DIN), lambda i: (0, 0)),
            pl.BlockSpec((1, DIN), lambda i: (0, 0)),
        ],
        out_specs=pl.BlockSpec((GG, DIN), lambda i: (0, 0)),
        out_shape=jax.ShapeDtypeStruct((GG, DIN), jnp.float32),
        scratch_shapes=[pltpu.VMEM((GG, HH), jnp.float32)],
        compiler_params=_ARB,
    )(v, s, t, batch3, Wf, bf)


def kernel(x, edge_index, batch, W1, b1, g1, be1, W2, b2, g2, be2, W3, b3, g3, be3, W4, b4, g4, be4, W5, b5, g5, be5, Wf, bf):
    src = edge_index[0].astype(jnp.int32)
    dst = edge_index[1].astype(jnp.int32)
    src_p = jnp.concatenate([src, jnp.zeros((E_PAD - EE,), jnp.int32)])
    dst_p = jnp.concatenate([dst, jnp.full((E_PAD - EE,), NN, jnp.int32)])
    c2 = jnp.arange(2, dtype=jnp.int32) * NN
    c8 = jnp.arange(8, dtype=jnp.int32) * NN
    src_c2 = (src_p[None, :] + c2[:, None]).reshape(-1)
    src_c8 = (src_p[None, :] + c8[:, None]).reshape(-1)
    zero_slab = jnp.zeros((RPT, LC), jnp.float32)
    batch3 = batch.astype(jnp.int32).reshape(NT, 1, ROW_TILE)

    def vec(p):
        return p.reshape(HH // LC, LC)

    x2 = x.reshape(NN, DIN // LC, LC).transpose(1, 0, 2)
    aggx = _sc_agg_call(x2.reshape(-1, LC), src_c2, dst_p, zero_slab, 2)
    z = _call_m1(x, aggx.reshape(DIN // LC, NN, LC), W1)
    v, s, t = _call_ep(z, None, vec(b1), vec(g1), vec(be1))

    for (W, b, g, be) in ((W2, b2, g2, be2), (W3, b3, g3, be3),
                          (W4, b4, g4, be4), (W5, b5, g5, be5)):
        z = _call_mm(v, s.reshape(1, HH), t.reshape(1, HH), W)
        a = _sc_agg_call(z.reshape(-1, LC), src_c8, dst_p, zero_slab, 8)
        v, s, t = _call_ep(z, a.reshape(HH // LC, NN, LC), vec(b),
                           vec(g), vec(be))

    return _call_pool(v, s.reshape(1, HH), t.reshape(1, HH), batch3, Wf, bf.reshape(1, DIN))
